# baseline (speedup 1.0000x reference)
import functools

import jax
import jax.numpy as jnp
from jax import lax
from jax.experimental import pallas as pl
from jax.experimental.pallas import tpu as pltpu
from jax.experimental.pallas import tpu_sc as plsc

B = 16384
D = 16
NC = 2
NS = 16
NW = NC * NS
NV = 1000000
NL = 100000
NJ = (NV + 127) // 128
JPW = (NJ + NW - 1) // NW
CAP = 768
CAPP = CAP + 16
CAPV = CAP // 16
NCH = CAP // 128
NZG = 16

_mesh = plsc.VectorSubcoreMesh(core_axis_name="c", subcore_axis_name="s")


def _rsqrt_s(x):
    i = lax.bitcast_convert_type(x, jnp.int32)
    i = jnp.int32(0x5F3759DF) - lax.shift_right_arithmetic(i, 1)
    y = lax.bitcast_convert_type(i, jnp.float32)
    for _ in range(3):
        y = y * (jnp.float32(1.5) - jnp.float32(0.5) * x * y * y)
    return y


@functools.partial(
    pl.kernel,
    mesh=_mesh,
    compiler_params=pltpu.CompilerParams(
        needs_layout_passes=False, use_tc_tiling_on_sc=True),
    out_type=jax.ShapeDtypeStruct((B + 128,), jnp.float32),
    scratch_types=[
        pltpu.VMEM((128, 128), jnp.float32),
        pltpu.VMEM((CAP * 16,), jnp.float32),
        pltpu.VMEM((B,), jnp.int32),
        pltpu.VMEM((2048,), jnp.int32),
        pltpu.VMEM((2, 2, 8, 128), jnp.float32),
        pltpu.VMEM((2, 2, 8, 128), jnp.float32),
        pltpu.VMEM((CAPP,), jnp.int32),
        pltpu.VMEM((CAPP,), jnp.int32),
        pltpu.VMEM((CAP,), jnp.int32),
        pltpu.VMEM((CAP,), jnp.int32),
        pltpu.VMEM((CAP,), jnp.int32),
        pltpu.VMEM((CAP,), jnp.int32),
        pltpu.VMEM((CAP,), jnp.float32),
        pltpu.VMEM((CAP,), jnp.float32),
        pltpu.VMEM((CAP,), jnp.float32),
        pltpu.VMEM((CAP,), jnp.int32),
        pltpu.VMEM((272,), jnp.int32),
        pltpu.VMEM((272,), jnp.int32),
        pltpu.VMEM((272,), jnp.int32),
        pltpu.VMEM((NCH, 128), jnp.int32),
        pltpu.VMEM((16,), jnp.float32),
        pltpu.VMEM((16,), jnp.float32),
        pltpu.SemaphoreType.DMA,
        pltpu.SemaphoreType.DMA,
    ],
)
def _wnom_v2(legs_hbm, votes_hbm, ip8_hbm, ypT_hbm, npT_hbm, w_hbm, beta_hbm,
             out_hbm, a8, a16f, legsv, votp, ystg, nstg, rlist, ilist, rsrt,
             isrt, lvs, rgv, d1b, d2b, rvb, packv, histv, cursv, nzjv, sidx,
             wv, bv, sem, sem2):
    wid = lax.axis_index("s") * NC + lax.axis_index("c")
    jlo = wid * JPW
    jhi = jnp.minimum(jlo + JPW, NJ)
    rlo = jlo * 128
    rhi = jnp.minimum(jhi * 128, NV)

    pltpu.sync_copy(legs_hbm, legsv)
    pltpu.sync_copy(w_hbm, wv)
    pltpu.sync_copy(beta_hbm, bv)

    iota = lax.iota(jnp.int32, 16)
    zero16i = jnp.zeros((16,), jnp.int32)
    one16i = zero16i + 1

    def clr(j, _):
        histv[pl.ds(j * 16, 16)] = zero16i
        return 0

    lax.fori_loop(0, 17, clr, 0)

    def scan_piece(p, cnt):
        pltpu.sync_copy(votes_hbm.at[pl.ds(p * 2048, 2048)], votp)

        def scan_vreg(j, cnt2):
            r = votp[pl.ds(j * 16, 16)]
            m = (r >= rlo) & (r < rhi)
            ivec = p * 2048 + j * 16 + iota
            plsc.store_compressed(rlist.at[pl.ds(cnt2, 16)], r, mask=m)
            plsc.store_compressed(ilist.at[pl.ds(cnt2, 16)], ivec, mask=m)
            npop = plsc.all_reduce_population_count(m)[0]
            return jnp.minimum(cnt2 + npop, CAP)

        return lax.fori_loop(0, 128, scan_vreg, cnt)

    count = lax.fori_loop(0, 8, scan_piece, jnp.int32(0))

    def hist_vreg(j, _):
        @pl.when(j * 16 < count)
        def _():
            r = rlist[pl.ds(j * 16, 16)]
            jloc = lax.shift_right_logical(r, 7) - jlo
            m = (j * 16 + iota) < count
            jloc = jnp.where(m, jloc, 255)
            plsc.addupdate_scatter(histv, [jloc], one16i, mask=m)
        return 0

    lax.fori_loop(0, CAPV, hist_vreg, 0)

    def nzj_vreg(j, nnz):
        cnt = histv[pl.ds(j * 16, 16)]
        jj = j * 16 + iota
        m = (cnt > 0) & (jj < (jhi - jlo))
        packed = lax.shift_left(cnt, 16) + jj
        plsc.store_compressed(nzjv.at[pl.ds(nnz, 16)], packed, mask=m)
        return nnz + plsc.all_reduce_population_count(m)[0]

    nnz = lax.fori_loop(0, 16, nzj_vreg, jnp.int32(0))

    def pref(g, run):
        c16 = histv[pl.ds(g * 16, 16)]
        s16 = plsc.cumsum(c16)
        cursv[pl.ds(g * 16, 16)] = s16 - c16 + run
        return run + s16[15]

    lax.fori_loop(0, 16, pref, jnp.int32(0))

    def place_vreg(j, _):
        @pl.when(j * 16 < count)
        def _():
            rv = rlist[pl.ds(j * 16, 16)]
            iv = ilist[pl.ds(j * 16, 16)]
            m = (j * 16 + iota) < count
            jv = jnp.where(m, lax.shift_right_logical(rv, 7) - jlo, 255)
            base = plsc.load_gather(cursv, [jv])
            rank = zero16i
            for l in range(1, 16):
                eq = (jv == jv[l]) & (iota < l)
                npop = plsc.all_reduce_population_count(eq)[0]
                rank = jnp.where(iota == l, npop, rank)
            pos = jnp.where(m, base + rank, CAP - 1)
            plsc.store_scatter(rsrt, [pos], rv, mask=m)
            plsc.store_scatter(isrt, [pos], iv, mask=m)
            plsc.addupdate_scatter(cursv, [jv], one16i, mask=m)
        return 0

    lax.fori_loop(0, CAPV, place_vreg, 0)

    def legg(j, _):
        iv = isrt[pl.ds(j * 16, 16)]
        m = (j * 16 + iota) < count
        iv = jnp.where(m, iv, 0)
        lv = plsc.load_gather(legsv, [iv])
        lvs[pl.ds(j * 16, 16)] = lv
        rv = rsrt[pl.ds(j * 16, 16)]
        packv[pl.ds(j * 16, 16)] = (rv & 127) + lax.shift_left(lv & 7, 7)
        rgv[pl.ds(j * 16, 16)] = lax.shift_right_logical(lv, 3)
        return 0

    lax.fori_loop(0, CAPV, legg, 0)

    def agather(c, _):
        sl = pl.ds(c * 128, 128)
        pltpu.async_copy(ip8_hbm.at[rgv.at[sl]], a8, sem2).wait()

        def aext(k, _2):
            kk = c * 128 + k
            pk16 = plsc.load_gather(packv, [zero16i + kk])
            lm8 = lax.shift_right_logical(pk16, 7) & 7
            aidx = lax.shift_left(lm8, 4) + iota
            row = plsc.load_gather(a8, [zero16i + k, aidx])
            a16f[pl.ds(kk * 16, 16)] = row
            return 0

        lax.fori_loop(0, 128, aext, 0)
        return 0

    lax.fori_loop(0, NCH, agather, 0)

    w2v = wv[...] * wv[...]

    dqv = lax.shift_right_logical(iota, 3)
    dmv = iota & 7

    def jgrp(tg, k0):
        e16 = nzjv[pl.ds(tg * 16, 16)]
        k = k0
        for l in range(16):
            t = tg * 16 + l
            e = e16[l]
            jj = (e & 0xFFFF) + jlo
            cnt = lax.shift_right_logical(e, 16)
            live = t < nnz

            @pl.when(live)
            def _(jj=jj, cnt=cnt, k=k, t=t):
                bsel = t & 1
                cstart = pl.multiple_of(jj * 128, 128)
                cps = []
                for dq in range(2):
                    cps.append(pltpu.async_copy(
                        ypT_hbm.at[pl.ds(dq * 8, 8), pl.ds(cstart, 128)],
                        ystg.at[bsel, dq], sem))
                    cps.append(pltpu.async_copy(
                        npT_hbm.at[pl.ds(dq * 8, 8), pl.ds(cstart, 128)],
                        nstg.at[bsel, dq], sem))
                for cp in cps:
                    cp.wait()

                def match(k2, _):
                    pk16 = plsc.load_gather(packv, [zero16i + k2])
                    rmv = pk16 & 127
                    bv16 = zero16i + bsel
                    y = plsc.load_gather(ystg, [bv16, dqv, dmv, rmv])
                    n = plsc.load_gather(nstg, [bv16, dqv, dmv, rmv])
                    a = a16f[pl.ds(k2 * 16, 16)]
                    na = jnp.sum(a * a)
                    ny = jnp.sum(y * y)
                    nn = jnp.sum(n * n)
                    sa = jnp.minimum(jnp.float32(1.0), _rsqrt_s(na))
                    sy = jnp.minimum(jnp.float32(1.0), _rsqrt_s(ny))
                    sn = jnp.minimum(jnp.float32(1.0), _rsqrt_s(nn))
                    asc = a * sa
                    t1 = asc - y * sy
                    t2 = asc - n * sn
                    d1 = jnp.sum(t1 * t1 * w2v)
                    d2 = jnp.sum(t2 * t2 * w2v)
                    lane = k2 & 15
                    m16 = iota == lane
                    base = k2 - lane
                    d1b[pl.ds(base, 16)] = jnp.where(
                        m16, d1, d1b[pl.ds(base, 16)])
                    d2b[pl.ds(base, 16)] = jnp.where(
                        m16, d2, d2b[pl.ds(base, 16)])
                    return 0

                lax.fori_loop(k, k + cnt, match, 0)

            k = jnp.where(live, k + cnt, k)
        return k

    lax.fori_loop(0, NZG, jgrp, jnp.int32(0))

    beta16 = bv[...]

    for j in range(CAPV):
        d1 = d1b[pl.ds(j * 16, 16)]
        d2 = d2b[pl.ds(j * 16, 16)]
        r = beta16 * (jnp.exp(jnp.float32(-0.5) * d1)
                      - jnp.exp(jnp.float32(-0.5) * d2))
        rvb[pl.ds(j * 16, 16)] = r
        m = (j * 16 + iota) < count
        iv = jnp.where(m, isrt[pl.ds(j * 16, 16)], B + (wid & 127))
        sidx[j // 8, pl.ds((j % 8) * 16, 16)] = iv

    scps = []
    for c in range(NCH):
        sl = pl.ds(c * 128, 128)
        scps.append(pltpu.async_copy(rvb.at[sl], out_hbm.at[sidx.at[c]],
                                     sem2))
    for cp in scps:
        cp.wait()


def kernel(legs, votes, ideal_points, yes_points, no_points, w, beta):
    legs32 = legs.astype(jnp.int32)
    votes32 = votes.astype(jnp.int32)
    ip8 = ideal_points.astype(jnp.float32).reshape(NL * D // 128, 128)
    ypT = yes_points.astype(jnp.float32).T
    npT = no_points.astype(jnp.float32).T
    beta16 = jnp.broadcast_to(beta.astype(jnp.float32), (D,))
    out = _wnom_v2(legs32, votes32, ip8, ypT, npT,
                   w.astype(jnp.float32), beta16)
    return out[:B]
